# baseline (speedup 1.0000x reference)
import functools

import jax
import jax.numpy as jnp
from jax import lax
from jax.experimental import pallas as pl
from jax.experimental.pallas import tpu as pltpu
from jax.experimental.pallas import tpu_sc as plsc

BATCH = 16384
EMBED_DIM = 32
NUM_TOPICS = 1000000
NUM_CORES = 2
NUM_SUBCORES = 16
NUM_WORKERS = NUM_CORES * NUM_SUBCORES
B_PER_W = BATCH // NUM_WORKERS
SUB = 64
N_SUB = B_PER_W // SUB

_MESH = plsc.VectorSubcoreMesh(core_axis_name="c", subcore_axis_name="s")


@functools.partial(
    pl.kernel,
    mesh=_MESH,
    out_type=jax.ShapeDtypeStruct((BATCH, EMBED_DIM), jnp.float32),
    scratch_types=[
        pltpu.VMEM((N_SUB, SUB), jnp.int32),
        pltpu.VMEM((B_PER_W,), jnp.int32),
        pltpu.VMEM((SUB, 128), jnp.float32),
        pltpu.VMEM((B_PER_W, EMBED_DIM), jnp.float32),
        pltpu.SemaphoreType.DMA,
    ],
    compiler_params=pltpu.CompilerParams(needs_layout_passes=False),
)
def _sc_gather(hi_hbm, off_hbm, table_hbm, out_hbm, hi_v, off_v, staged,
               rows_v, sem):
    wid = lax.axis_index("s") * NUM_CORES + lax.axis_index("c")
    base = wid * B_PER_W
    pltpu.sync_copy(hi_hbm.at[wid], hi_v)
    pltpu.sync_copy(off_hbm.at[wid], off_v)
    lanes = lax.iota(jnp.int32, 16)

    @pl.loop(0, N_SUB)
    def _sub(k):
        pltpu.async_copy(table_hbm.at[hi_v.at[k]], staged, sem).wait()
        for j in range(SUB // 16):
            rows16 = lanes + (j * 16)
            offs16 = off_v[pl.ds(k * SUB + j * 16, 16)]
            grows16 = rows16 + k * SUB
            for c in range(EMBED_DIM):
                col = jnp.full((16,), c, jnp.int32)
                v = plsc.load_gather(staged, [rows16, offs16 + c])
                plsc.store_scatter(rows_v, [grows16, col], v)

    pltpu.sync_copy(rows_v, out_hbm.at[pl.ds(base, B_PER_W)])


def kernel(topic_labels, embedding_weight):
    idx = topic_labels.astype(jnp.int32)
    hi = (idx >> 2).reshape(NUM_WORKERS, N_SUB, SUB)
    off = ((idx & 3) << 5).reshape(NUM_WORKERS, B_PER_W)
    table2 = embedding_weight.reshape(NUM_TOPICS // 4, 128)
    return _sc_gather(hi, off, table2)


# speedup vs baseline: 1.0505x; 1.0505x over previous
import functools

import jax
import jax.numpy as jnp
from jax import lax
from jax.experimental import pallas as pl
from jax.experimental.pallas import tpu as pltpu
from jax.experimental.pallas import tpu_sc as plsc

BATCH = 16384
EMBED_DIM = 32
NUM_TOPICS = 1000000
NUM_CORES = 2
NUM_SUBCORES = 16
NUM_WORKERS = NUM_CORES * NUM_SUBCORES
B_PER_W = BATCH // NUM_WORKERS
CHUNK = 128
N_CHUNKS = B_PER_W // CHUNK

_MESH = plsc.VectorSubcoreMesh(core_axis_name="c", subcore_axis_name="s")


@functools.partial(
    pl.kernel,
    mesh=_MESH,
    out_type=jax.ShapeDtypeStruct((BATCH, 128), jnp.float32),
    scratch_types=[
        pltpu.VMEM((N_CHUNKS, CHUNK), jnp.int32),
        pltpu.VMEM((CHUNK, EMBED_DIM), jnp.float32),
        pltpu.SemaphoreType.DMA,
    ],
    compiler_params=pltpu.CompilerParams(
        needs_layout_passes=False,
        use_tc_tiling_on_sc=False,
        disable_bounds_checks=True,
    ),
)
def _sc_gather(idx_hbm, table_hbm, out_hbm, idx_v, staged, sem):
    wid = lax.axis_index("s") * NUM_CORES + lax.axis_index("c")
    base = wid * B_PER_W
    pltpu.sync_copy(idx_hbm.at[pl.ds(wid * N_CHUNKS, N_CHUNKS)], idx_v)

    @pl.loop(0, N_CHUNKS)
    def _chunk(j):
        pltpu.async_copy(table_hbm.at[idx_v.at[j]], staged, sem).wait()
        pltpu.sync_copy(
            staged,
            out_hbm.at[pl.ds(base + j * CHUNK, CHUNK), pl.ds(0, EMBED_DIM)],
        )


def kernel(topic_labels, embedding_weight):
    idx4 = topic_labels.astype(jnp.int32).reshape(BATCH // 128, 128)
    out128 = _sc_gather(idx4, embedding_weight)
    return out128[:, :EMBED_DIM]


# speedup vs baseline: 1.0564x; 1.0056x over previous
import functools

import jax
import jax.numpy as jnp
from jax import lax
from jax.experimental import pallas as pl
from jax.experimental.pallas import tpu as pltpu
from jax.experimental.pallas import tpu_sc as plsc

BATCH = 16384
EMBED_DIM = 32
NUM_TOPICS = 1000000
NUM_CORES = 2
NUM_SUBCORES = 16
NUM_WORKERS = NUM_CORES * NUM_SUBCORES
B_PER_W = BATCH // NUM_WORKERS
CHUNK = 128
N_CHUNKS = B_PER_W // CHUNK

_MESH = plsc.VectorSubcoreMesh(core_axis_name="c", subcore_axis_name="s")


@functools.partial(
    pl.kernel,
    mesh=_MESH,
    out_type=jax.ShapeDtypeStruct((BATCH, 128), jnp.float32),
    scratch_types=[
        pltpu.VMEM((N_CHUNKS, CHUNK), jnp.int32),
        pltpu.VMEM((CHUNK, EMBED_DIM), jnp.float32),
        pltpu.SemaphoreType.DMA,
    ],
    compiler_params=pltpu.CompilerParams(
        needs_layout_passes=False,
        use_tc_tiling_on_sc=False,
        disable_bounds_checks=True,
    ),
)
def _sc_gather(idx_hbm, table_hbm, out_hbm, idx_v, staged, sem):
    wid = lax.axis_index("s") * NUM_CORES + lax.axis_index("c")
    base = wid * B_PER_W
    pltpu.sync_copy(idx_hbm.at[pl.ds(wid * N_CHUNKS, N_CHUNKS)], idx_v)

    @pl.loop(0, N_CHUNKS)
    def _chunk(j):
        pltpu.async_copy(table_hbm.at[idx_v.at[j]], staged, sem).wait()
        pltpu.sync_copy(
            staged,
            out_hbm.at[pl.ds(base + j * CHUNK, CHUNK), pl.ds(0, EMBED_DIM)],
        )


def kernel(topic_labels, embedding_weight):
    idx = topic_labels.astype(jnp.int32).reshape(BATCH // 128, 128)
    out128 = _sc_gather(idx, embedding_weight)
    return out128[:, :EMBED_DIM]
